# baseline (speedup 1.0000x reference)
import functools

import jax
import jax.numpy as jnp
from jax import lax
from jax.experimental import pallas as pl
from jax.experimental.pallas import tpu as pltpu
from jax.experimental.pallas import tpu_sc as plsc

_B = 64
_LF = 1025
_D = 192
_K = 512
_G = 128
_NCH = _K // _G
_NW = 32
_BPW = _B // _NW


def _flat_keep_indices():
    rand = jax.random.normal(jax.random.key(42), (_B, _LF - 1), jnp.float32)
    _, keep = lax.top_k(rand, _K)
    base = (jnp.arange(_B, dtype=jnp.int32) * _LF)[:, None]
    flat = base + keep.astype(jnp.int32) + 1
    return flat.reshape(_B * _K)


@functools.lru_cache(maxsize=1)
def _build():
    mesh = plsc.VectorSubcoreMesh(core_axis_name="c", subcore_axis_name="s")

    @functools.partial(
        pl.kernel,
        mesh=mesh,
        compiler_params=pltpu.CompilerParams(use_tc_tiling_on_sc=False),
        out_type=jax.ShapeDtypeStruct((_B * (1 + _K), _D), jnp.float32),
        scratch_types=[
            pltpu.VMEM((_K,), jnp.int32),
            pltpu.VMEM((_K, _D), jnp.float32),
            pltpu.SemaphoreType.DMA,
        ],
    )
    def gather_kernel(x_hbm, idx_hbm, out_hbm, idx_v, buf, sem):
        wid = lax.axis_index("s") * 2 + lax.axis_index("c")
        for bb in range(_BPW):
            b = wid * _BPW + bb
            pltpu.sync_copy(idx_hbm.at[pl.ds(b * _K, _K)], idx_v)
            pltpu.sync_copy(
                x_hbm.at[pl.ds(b * _LF, 1)],
                out_hbm.at[pl.ds(b * (1 + _K), 1)],
            )
            copies = [
                pltpu.async_copy(
                    x_hbm.at[idx_v.at[pl.ds(t * _G, _G)]],
                    buf.at[pl.ds(t * _G, _G)],
                    sem,
                )
                for t in range(_NCH)
            ]
            for c in copies:
                c.wait()
            pltpu.sync_copy(buf, out_hbm.at[pl.ds(b * (1 + _K) + 1, _K)])

    return gather_kernel


def kernel(x):
    idx = _flat_keep_indices()
    out = _build()(x.reshape(_B * _LF, _D), idx)
    return out.reshape(_B, 1 + _K, _D)


# speedup vs baseline: 2.8508x; 2.8508x over previous
import functools

import jax
import jax.numpy as jnp
from jax import lax
from jax.experimental import pallas as pl
from jax.experimental.pallas import tpu as pltpu
from jax.experimental.pallas import tpu_sc as plsc

_B = 64
_LF = 1025
_D = 192
_KP = 513
_DT = _D // 8
_NG = 32
_IPAD = 640


def _tok_indices():
    rand = jax.random.normal(jax.random.key(42), (_B, _LF - 1), jnp.float32)
    _, keep = lax.top_k(rand, _KP - 1)
    tok = jnp.concatenate(
        [
            jnp.zeros((_B, 1), jnp.int32),
            keep.astype(jnp.int32) + 1,
            jnp.zeros((_B, _IPAD - _KP), jnp.int32),
        ],
        axis=1,
    )
    return tok.reshape(_B * _IPAD)


@functools.lru_cache(maxsize=1)
def _build():
    mesh = plsc.VectorSubcoreMesh(core_axis_name="c", subcore_axis_name="s")

    @functools.partial(
        pl.kernel,
        mesh=mesh,
        compiler_params=pltpu.CompilerParams(
            use_tc_tiling_on_sc=True, needs_layout_passes=False
        ),
        out_type=jax.ShapeDtypeStruct((_B, _D, _KP), jnp.float32),
        scratch_types=[
            pltpu.VMEM((_IPAD,), jnp.int32),
            pltpu.VMEM((8, _LF), jnp.float32),
            pltpu.VMEM((8, _KP), jnp.float32),
        ],
    )
    def gather_kernel(x_hbm, tok_hbm, out_hbm, idx_v, slab, oslab):
        wid = lax.axis_index("s") * 2 + lax.axis_index("c")
        lane = lax.iota(jnp.int32, 16)
        mask0 = lane == 0
        for bb in range(2):
            b = wid * 2 + bb
            pltpu.sync_copy(tok_hbm.at[pl.ds(b * _IPAD, _IPAD)], idx_v)

            def dt_body(dt, carry):
                r0 = pl.multiple_of(dt * 8, 8)
                pltpu.sync_copy(x_hbm.at[b, pl.ds(r0, 8), :], slab)

                def g_body(g, c):
                    j0 = pl.multiple_of(g * 16, 16)
                    tok_vec = idx_v[pl.ds(j0, 16)]
                    l_out = lane + j0
                    for s in range(8):
                        svec = jnp.full((16,), s, jnp.int32)
                        vals = plsc.load_gather(slab, [svec, tok_vec])
                        plsc.store_scatter(oslab, [svec, l_out], vals)
                    return c

                lax.fori_loop(0, _NG, g_body, 0, unroll=False)
                tok_tail = idx_v[pl.ds(_KP - 1, 16)]
                l_tail = jnp.full((16,), _KP - 1, jnp.int32)
                for s in range(8):
                    svec = jnp.full((16,), s, jnp.int32)
                    vals = plsc.load_gather(slab, [svec, tok_tail])
                    plsc.store_scatter(oslab, [svec, l_tail], vals, mask=mask0)
                pltpu.sync_copy(oslab, out_hbm.at[b, pl.ds(r0, 8), :])
                return carry

            lax.fori_loop(0, _DT, dt_body, 0, unroll=False)

    return gather_kernel


def kernel(x):
    tok = _tok_indices()
    out_t = _build()(x.transpose(0, 2, 1), tok)
    return out_t.transpose(0, 2, 1)


# speedup vs baseline: 5.3534x; 1.8779x over previous
import functools

import jax
import jax.numpy as jnp
import numpy as np
from jax import lax
from jax.experimental import pallas as pl
from jax.experimental.pallas import tpu as pltpu
from jax.experimental.pallas import tpu_sc as plsc

_B = 64
_LF = 1025
_D = 192
_KP = 513
_DT = _D // 8
_NG = 32
_IPAD = 640


def _threefry_bits(k1, k2, n):
    x0 = np.zeros(n, np.uint32)
    x1 = np.arange(n, dtype=np.uint32)
    rotations = [(13, 15, 26, 6), (17, 29, 16, 24)]
    ks = [np.uint32(k1), np.uint32(k2),
          np.uint32(k1) ^ np.uint32(k2) ^ np.uint32(0x1BD11BDA)]

    def rounds(x0, x1, rs):
        for r in rs:
            x0 = (x0 + x1).astype(np.uint32)
            x1 = ((x1 << np.uint32(r)) | (x1 >> np.uint32(32 - r))).astype(
                np.uint32) ^ x0
        return x0, x1

    x0 = (x0 + ks[0]).astype(np.uint32)
    x1 = (x1 + ks[1]).astype(np.uint32)
    add = [(ks[1], ks[2], 1), (ks[2], ks[0], 2), (ks[0], ks[1], 3),
           (ks[1], ks[2], 4), (ks[2], ks[0], 5)]
    for i, (a0, a1, c) in enumerate(add):
        x0, x1 = rounds(x0, x1, rotations[i % 2])
        x0 = (x0 + a0).astype(np.uint32)
        x1 = (x1 + a1 + np.uint32(c)).astype(np.uint32)
    return x0 ^ x1


def _tok_indices_np():
    vals = (_threefry_bits(0, 42, _B * (_LF - 1)) >> np.uint32(9))
    vals = vals.reshape(_B, _LF - 1)
    keep = np.argsort(-vals.astype(np.int64), axis=1, kind="stable")
    keep = keep[:, : _KP - 1].astype(np.int32)
    tok = np.zeros((_B, _IPAD), np.int32)
    tok[:, 1:_KP] = keep + 1
    return tok.reshape(_B * _IPAD)


_TOK = _tok_indices_np()


@functools.lru_cache(maxsize=1)
def _build():
    mesh = plsc.VectorSubcoreMesh(core_axis_name="c", subcore_axis_name="s")

    @functools.partial(
        pl.kernel,
        mesh=mesh,
        compiler_params=pltpu.CompilerParams(
            use_tc_tiling_on_sc=True, needs_layout_passes=False
        ),
        out_type=jax.ShapeDtypeStruct((_B, _D, _KP), jnp.float32),
        scratch_types=[
            pltpu.VMEM((_IPAD,), jnp.int32),
            pltpu.VMEM((8, _LF), jnp.float32),
            pltpu.VMEM((8, _LF), jnp.float32),
            pltpu.VMEM((8, _KP), jnp.float32),
            pltpu.VMEM((8, _KP), jnp.float32),
            pltpu.SemaphoreType.DMA,
            pltpu.SemaphoreType.DMA,
            pltpu.SemaphoreType.DMA,
            pltpu.SemaphoreType.DMA,
        ],
    )
    def gather_kernel(
        x_hbm, tok_hbm, out_hbm,
        idx_v, slab_a, slab_b, oslab_a, oslab_b, s_ia, s_ib, s_oa, s_ob,
    ):
        wid = lax.axis_index("s") * 2 + lax.axis_index("c")
        lane = lax.iota(jnp.int32, 16)
        mask0 = lane == 0

        def in_copy(b, dt, slab, sem):
            r0 = pl.multiple_of(dt * 8, 8)
            return pltpu.make_async_copy(x_hbm.at[b, pl.ds(r0, 8), :], slab, sem)

        def out_copy(b, dt, oslab, sem):
            r0 = pl.multiple_of(dt * 8, 8)
            return pltpu.make_async_copy(oslab, out_hbm.at[b, pl.ds(r0, 8), :], sem)

        def compute(slab, oslab):
            def g2_body(h, c):
                for u in range(2):
                    j0 = pl.multiple_of((h * 2 + u) * 16, 16)
                    tok_vec = idx_v[pl.ds(j0, 16)]
                    l_out = lane + j0
                    for s in range(8):
                        svec = jnp.full((16,), s, jnp.int32)
                        vals = plsc.load_gather(slab, [svec, tok_vec])
                        plsc.store_scatter(oslab, [svec, l_out], vals)
                return c

            lax.fori_loop(0, _NG // 2, g2_body, 0, unroll=False)
            tok_tail = idx_v[pl.ds(_KP - 1, 16)]
            l_tail = jnp.full((16,), _KP - 1, jnp.int32)
            for s in range(8):
                svec = jnp.full((16,), s, jnp.int32)
                vals = plsc.load_gather(slab, [svec, tok_tail])
                plsc.store_scatter(oslab, [svec, l_tail], vals, mask=mask0)

        for bb in range(2):
            b = wid * 2 + bb
            pltpu.sync_copy(tok_hbm.at[pl.ds(b * _IPAD, _IPAD)], idx_v)
            in_copy(b, 0, slab_a, s_ia).start()

            def i_body(i, c):
                dt_a = i * 2
                dt_b = dt_a + 1
                in_copy(b, dt_a, slab_a, s_ia).wait()
                in_copy(b, dt_b, slab_b, s_ib).start()

                @pl.when(i > 0)
                def _():
                    out_copy(b, dt_a - 2, oslab_a, s_oa).wait()

                compute(slab_a, oslab_a)
                out_copy(b, dt_a, oslab_a, s_oa).start()

                in_copy(b, dt_b, slab_b, s_ib).wait()
                nxt = jnp.minimum(dt_b + 1, _DT - 1)
                in_copy(b, nxt, slab_a, s_ia).start()

                @pl.when(i > 0)
                def _():
                    out_copy(b, dt_b - 2, oslab_b, s_ob).wait()

                compute(slab_b, oslab_b)
                out_copy(b, dt_b, oslab_b, s_ob).start()
                return c

            lax.fori_loop(0, _DT // 2, i_body, 0, unroll=False)
            in_copy(b, _DT - 1, slab_a, s_ia).wait()
            out_copy(b, _DT - 2, oslab_a, s_oa).wait()
            out_copy(b, _DT - 1, oslab_b, s_ob).wait()

    return gather_kernel


def kernel(x):
    out_t = _build()(x.transpose(0, 2, 1), _TOK)
    return out_t.transpose(0, 2, 1)


# speedup vs baseline: 5.3623x; 1.0017x over previous
import functools

import jax
import jax.numpy as jnp
import numpy as np
from jax import lax
from jax.experimental import pallas as pl
from jax.experimental.pallas import tpu as pltpu
from jax.experimental.pallas import tpu_sc as plsc

_B = 64
_LF = 1025
_D = 192
_KP = 513
_DT = _D // 8
_NG = 32
_IPAD = 640


def _threefry_bits(k1, k2, n):
    x0 = np.zeros(n, np.uint32)
    x1 = np.arange(n, dtype=np.uint32)
    rotations = [(13, 15, 26, 6), (17, 29, 16, 24)]
    ks = [np.uint32(k1), np.uint32(k2),
          np.uint32(k1) ^ np.uint32(k2) ^ np.uint32(0x1BD11BDA)]

    def rounds(x0, x1, rs):
        for r in rs:
            x0 = (x0 + x1).astype(np.uint32)
            x1 = ((x1 << np.uint32(r)) | (x1 >> np.uint32(32 - r))).astype(
                np.uint32) ^ x0
        return x0, x1

    x0 = (x0 + ks[0]).astype(np.uint32)
    x1 = (x1 + ks[1]).astype(np.uint32)
    add = [(ks[1], ks[2], 1), (ks[2], ks[0], 2), (ks[0], ks[1], 3),
           (ks[1], ks[2], 4), (ks[2], ks[0], 5)]
    for i, (a0, a1, c) in enumerate(add):
        x0, x1 = rounds(x0, x1, rotations[i % 2])
        x0 = (x0 + a0).astype(np.uint32)
        x1 = (x1 + a1 + np.uint32(c)).astype(np.uint32)
    return x0 ^ x1


def _tok_indices_np():
    vals = (_threefry_bits(0, 42, _B * (_LF - 1)) >> np.uint32(9))
    vals = vals.reshape(_B, _LF - 1)
    keep = np.argsort(-vals.astype(np.int64), axis=1, kind="stable")
    keep = keep[:, : _KP - 1].astype(np.int32)
    tok = np.zeros((_B, _IPAD), np.int32)
    tok[:, 1:_KP] = keep + 1
    return tok.reshape(_B * _IPAD)


_TOK = _tok_indices_np()


@functools.lru_cache(maxsize=1)
def _build():
    mesh = plsc.VectorSubcoreMesh(core_axis_name="c", subcore_axis_name="s")

    @functools.partial(
        pl.kernel,
        mesh=mesh,
        compiler_params=pltpu.CompilerParams(
            use_tc_tiling_on_sc=True, needs_layout_passes=False
        ),
        out_type=jax.ShapeDtypeStruct((_B, _D, _KP), jnp.float32),
        scratch_types=[
            pltpu.VMEM((_IPAD,), jnp.int32),
            pltpu.VMEM((8, _LF), jnp.float32),
            pltpu.VMEM((8, _LF), jnp.float32),
            pltpu.VMEM((8, _KP), jnp.float32),
            pltpu.VMEM((8, _KP), jnp.float32),
            pltpu.SemaphoreType.DMA,
            pltpu.SemaphoreType.DMA,
            pltpu.SemaphoreType.DMA,
            pltpu.SemaphoreType.DMA,
        ],
    )
    def gather_kernel(
        x_hbm, tok_hbm, out_hbm,
        idx_v, slab_a, slab_b, oslab_a, oslab_b, s_ia, s_ib, s_oa, s_ob,
    ):
        wid = lax.axis_index("s") * 2 + lax.axis_index("c")
        lane = lax.iota(jnp.int32, 16)
        mask0 = lane == 0

        def in_copy(b, dt, slab, sem):
            r0 = pl.multiple_of(dt * 8, 8)
            return pltpu.make_async_copy(x_hbm.at[b, pl.ds(r0, 8), :], slab, sem)

        def out_copy(b, dt, oslab, sem):
            r0 = pl.multiple_of(dt * 8, 8)
            return pltpu.make_async_copy(oslab, out_hbm.at[b, pl.ds(r0, 8), :], sem)

        def compute(slab, oslab):
            def g2_body(h, c):
                for u in range(2):
                    j0 = pl.multiple_of((h * 2 + u) * 16, 16)
                    tok_vec = idx_v[pl.ds(j0, 16)]
                    for s in range(8):
                        svec = jnp.full((16,), s, jnp.int32)
                        vals = plsc.load_gather(slab, [svec, tok_vec])
                        oslab[s, pl.ds(j0, 16)] = vals
                return c

            lax.fori_loop(0, _NG // 2, g2_body, 0, unroll=False)
            tok_tail = idx_v[pl.ds(_KP - 1, 16)]
            l_tail = jnp.full((16,), _KP - 1, jnp.int32)
            for s in range(8):
                svec = jnp.full((16,), s, jnp.int32)
                vals = plsc.load_gather(slab, [svec, tok_tail])
                plsc.store_scatter(oslab, [svec, l_tail], vals, mask=mask0)

        for bb in range(2):
            b = wid * 2 + bb
            pltpu.sync_copy(tok_hbm.at[pl.ds(b * _IPAD, _IPAD)], idx_v)
            in_copy(b, 0, slab_a, s_ia).start()

            def i_body(i, c):
                dt_a = i * 2
                dt_b = dt_a + 1
                in_copy(b, dt_a, slab_a, s_ia).wait()
                in_copy(b, dt_b, slab_b, s_ib).start()

                @pl.when(i > 0)
                def _():
                    out_copy(b, dt_a - 2, oslab_a, s_oa).wait()

                compute(slab_a, oslab_a)
                out_copy(b, dt_a, oslab_a, s_oa).start()

                in_copy(b, dt_b, slab_b, s_ib).wait()
                nxt = jnp.minimum(dt_b + 1, _DT - 1)
                in_copy(b, nxt, slab_a, s_ia).start()

                @pl.when(i > 0)
                def _():
                    out_copy(b, dt_b - 2, oslab_b, s_ob).wait()

                compute(slab_b, oslab_b)
                out_copy(b, dt_b, oslab_b, s_ob).start()
                return c

            lax.fori_loop(0, _DT // 2, i_body, 0, unroll=False)
            in_copy(b, _DT - 1, slab_a, s_ia).wait()
            out_copy(b, _DT - 2, oslab_a, s_oa).wait()
            out_copy(b, _DT - 1, oslab_b, s_ob).wait()

    return gather_kernel


def kernel(x):
    out_t = _build()(x.transpose(0, 2, 1), _TOK)
    return out_t.transpose(0, 2, 1)


# speedup vs baseline: 6.0489x; 1.1280x over previous
import functools

import jax
import jax.numpy as jnp
import numpy as np
from jax import lax
from jax.experimental import pallas as pl
from jax.experimental.pallas import tpu as pltpu
from jax.experimental.pallas import tpu_sc as plsc

_B = 64
_LF = 1025
_D = 192
_KP = 513
_DT = _D // 8
_NG = 32
_IPAD = 640


def _threefry_bits(k1, k2, n):
    x0 = np.zeros(n, np.uint32)
    x1 = np.arange(n, dtype=np.uint32)
    rotations = [(13, 15, 26, 6), (17, 29, 16, 24)]
    ks = [np.uint32(k1), np.uint32(k2),
          np.uint32(k1) ^ np.uint32(k2) ^ np.uint32(0x1BD11BDA)]

    def rounds(x0, x1, rs):
        for r in rs:
            x0 = (x0 + x1).astype(np.uint32)
            x1 = ((x1 << np.uint32(r)) | (x1 >> np.uint32(32 - r))).astype(
                np.uint32) ^ x0
        return x0, x1

    x0 = (x0 + ks[0]).astype(np.uint32)
    x1 = (x1 + ks[1]).astype(np.uint32)
    add = [(ks[1], ks[2], 1), (ks[2], ks[0], 2), (ks[0], ks[1], 3),
           (ks[1], ks[2], 4), (ks[2], ks[0], 5)]
    for i, (a0, a1, c) in enumerate(add):
        x0, x1 = rounds(x0, x1, rotations[i % 2])
        x0 = (x0 + a0).astype(np.uint32)
        x1 = (x1 + a1 + np.uint32(c)).astype(np.uint32)
    return x0 ^ x1


def _tok_indices_np():
    vals = (_threefry_bits(0, 42, _B * (_LF - 1)) >> np.uint32(9))
    vals = vals.reshape(_B, _LF - 1)
    keep = np.argsort(-vals.astype(np.int64), axis=1, kind="stable")
    keep = keep[:, : _KP - 1].astype(np.int32)
    tok = np.zeros((_B, _IPAD), np.int32)
    tok[:, 1:_KP] = keep + 1
    return tok.reshape(_B * _IPAD)


_TOK = _tok_indices_np()


@functools.lru_cache(maxsize=1)
def _build():
    mesh = plsc.VectorSubcoreMesh(core_axis_name="c", subcore_axis_name="s")

    @functools.partial(
        pl.kernel,
        mesh=mesh,
        compiler_params=pltpu.CompilerParams(
            use_tc_tiling_on_sc=True, needs_layout_passes=False
        ),
        out_type=jax.ShapeDtypeStruct((_B, _D, _KP), jnp.float32),
        scratch_types=[
            pltpu.VMEM((_IPAD,), jnp.int32),
            pltpu.VMEM((8, _LF), jnp.float32),
            pltpu.VMEM((8, _LF), jnp.float32),
            pltpu.VMEM((8, _KP), jnp.float32),
            pltpu.VMEM((8, _KP), jnp.float32),
            pltpu.SemaphoreType.DMA,
            pltpu.SemaphoreType.DMA,
            pltpu.SemaphoreType.DMA,
            pltpu.SemaphoreType.DMA,
        ],
    )
    def gather_kernel(
        x_hbm, tok_hbm, out_hbm,
        idx_v, slab_a, slab_b, oslab_a, oslab_b, s_ia, s_ib, s_oa, s_ob,
    ):
        wid = lax.axis_index("s") * 2 + lax.axis_index("c")
        lane = lax.iota(jnp.int32, 16)
        mask0 = lane == 0

        def in_copy(b, dt, slab, sem):
            r0 = pl.multiple_of(dt * 8, 8)
            return pltpu.make_async_copy(x_hbm.at[b, pl.ds(r0, 8), :], slab, sem)

        def out_copy(b, dt, oslab, sem):
            r0 = pl.multiple_of(dt * 8, 8)
            return pltpu.make_async_copy(oslab, out_hbm.at[b, pl.ds(r0, 8), :], sem)

        def compute(slab, oslab):
            @plsc.parallel_loop(0, _NG, step=1, unroll=4)
            def _(g):
                j0 = pl.multiple_of(g * 16, 16)
                tok_vec = idx_v[pl.ds(j0, 16)]
                for s in range(8):
                    svec = jnp.full((16,), s, jnp.int32)
                    vals = plsc.load_gather(slab, [svec, tok_vec])
                    oslab[s, pl.ds(j0, 16)] = vals
            tok_tail = idx_v[pl.ds(_KP - 1, 16)]
            l_tail = jnp.full((16,), _KP - 1, jnp.int32)
            for s in range(8):
                svec = jnp.full((16,), s, jnp.int32)
                vals = plsc.load_gather(slab, [svec, tok_tail])
                plsc.store_scatter(oslab, [svec, l_tail], vals, mask=mask0)

        for bb in range(2):
            b = wid * 2 + bb
            pltpu.sync_copy(tok_hbm.at[pl.ds(b * _IPAD, _IPAD)], idx_v)
            in_copy(b, 0, slab_a, s_ia).start()

            def i_body(i, c):
                dt_a = i * 2
                dt_b = dt_a + 1
                in_copy(b, dt_a, slab_a, s_ia).wait()
                in_copy(b, dt_b, slab_b, s_ib).start()

                @pl.when(i > 0)
                def _():
                    out_copy(b, dt_a - 2, oslab_a, s_oa).wait()

                compute(slab_a, oslab_a)
                out_copy(b, dt_a, oslab_a, s_oa).start()

                in_copy(b, dt_b, slab_b, s_ib).wait()
                nxt = jnp.minimum(dt_b + 1, _DT - 1)
                in_copy(b, nxt, slab_a, s_ia).start()

                @pl.when(i > 0)
                def _():
                    out_copy(b, dt_b - 2, oslab_b, s_ob).wait()

                compute(slab_b, oslab_b)
                out_copy(b, dt_b, oslab_b, s_ob).start()
                return c

            lax.fori_loop(0, _DT // 2, i_body, 0, unroll=False)
            in_copy(b, _DT - 1, slab_a, s_ia).wait()
            out_copy(b, _DT - 2, oslab_a, s_oa).wait()
            out_copy(b, _DT - 1, oslab_b, s_ob).wait()

    return gather_kernel


def kernel(x):
    out_t = _build()(x.transpose(0, 2, 1), _TOK)
    return out_t.transpose(0, 2, 1)


# speedup vs baseline: 7.8421x; 1.2965x over previous
import functools

import jax
import jax.numpy as jnp
import numpy as np
from jax import lax
from jax.experimental import pallas as pl
from jax.experimental.pallas import tpu as pltpu
from jax.experimental.pallas import tpu_sc as plsc

_B = 64
_LF = 1025
_D = 192
_KP = 513
_SH = 24
_DT = _D // _SH
_NG = 32
_IPAD = 640


def _threefry_bits(k1, k2, n):
    x0 = np.zeros(n, np.uint32)
    x1 = np.arange(n, dtype=np.uint32)
    rotations = [(13, 15, 26, 6), (17, 29, 16, 24)]
    ks = [np.uint32(k1), np.uint32(k2),
          np.uint32(k1) ^ np.uint32(k2) ^ np.uint32(0x1BD11BDA)]

    def rounds(x0, x1, rs):
        for r in rs:
            x0 = (x0 + x1).astype(np.uint32)
            x1 = ((x1 << np.uint32(r)) | (x1 >> np.uint32(32 - r))).astype(
                np.uint32) ^ x0
        return x0, x1

    x0 = (x0 + ks[0]).astype(np.uint32)
    x1 = (x1 + ks[1]).astype(np.uint32)
    add = [(ks[1], ks[2], 1), (ks[2], ks[0], 2), (ks[0], ks[1], 3),
           (ks[1], ks[2], 4), (ks[2], ks[0], 5)]
    for i, (a0, a1, c) in enumerate(add):
        x0, x1 = rounds(x0, x1, rotations[i % 2])
        x0 = (x0 + a0).astype(np.uint32)
        x1 = (x1 + a1 + np.uint32(c)).astype(np.uint32)
    return x0 ^ x1


def _tok_indices_np():
    vals = (_threefry_bits(0, 42, _B * (_LF - 1)) >> np.uint32(9))
    vals = vals.reshape(_B, _LF - 1)
    keep = np.argsort(-vals.astype(np.int64), axis=1, kind="stable")
    keep = keep[:, : _KP - 1].astype(np.int32)
    tok = np.zeros((_B, _IPAD), np.int32)
    tok[:, 1:_KP] = keep + 1
    return tok.reshape(_B * _IPAD)


_TOK = _tok_indices_np()


@functools.lru_cache(maxsize=1)
def _build():
    mesh = plsc.VectorSubcoreMesh(core_axis_name="c", subcore_axis_name="s")

    @functools.partial(
        pl.kernel,
        mesh=mesh,
        compiler_params=pltpu.CompilerParams(
            use_tc_tiling_on_sc=True, needs_layout_passes=False
        ),
        out_type=jax.ShapeDtypeStruct((_B, _D, _KP), jnp.float32),
        scratch_types=[
            pltpu.VMEM((_IPAD,), jnp.int32),
            pltpu.VMEM((_SH, _LF), jnp.float32),
            pltpu.VMEM((_SH, _LF), jnp.float32),
            pltpu.VMEM((_SH, _KP), jnp.float32),
            pltpu.VMEM((_SH, _KP), jnp.float32),
            pltpu.SemaphoreType.DMA,
            pltpu.SemaphoreType.DMA,
            pltpu.SemaphoreType.DMA,
            pltpu.SemaphoreType.DMA,
        ],
    )
    def gather_kernel(
        x_hbm, tok_hbm, out_hbm,
        idx_v, slab_a, slab_b, oslab_a, oslab_b, s_ia, s_ib, s_oa, s_ob,
    ):
        wid = lax.axis_index("s") * 2 + lax.axis_index("c")
        lane = lax.iota(jnp.int32, 16)
        mask0 = lane == 0

        def in_copy(b, dt, slab, sem):
            r0 = pl.multiple_of(dt * _SH, 8)
            return pltpu.make_async_copy(x_hbm.at[b, pl.ds(r0, _SH), :], slab, sem)

        def out_copy(b, dt, oslab, sem):
            r0 = pl.multiple_of(dt * _SH, 8)
            return pltpu.make_async_copy(oslab, out_hbm.at[b, pl.ds(r0, _SH), :], sem)

        def compute(slab, oslab):
            @plsc.parallel_loop(0, _NG, step=1, unroll=2)
            def _(g):
                j0 = pl.multiple_of(g * 16, 16)
                tok_vec = idx_v[pl.ds(j0, 16)]
                for s in range(_SH):
                    svec = jnp.full((16,), s, jnp.int32)
                    vals = plsc.load_gather(slab, [svec, tok_vec])
                    oslab[s, pl.ds(j0, 16)] = vals
            tok_tail = idx_v[pl.ds(_KP - 1, 16)]
            l_tail = jnp.full((16,), _KP - 1, jnp.int32)
            for s in range(_SH):
                svec = jnp.full((16,), s, jnp.int32)
                vals = plsc.load_gather(slab, [svec, tok_tail])
                plsc.store_scatter(oslab, [svec, l_tail], vals, mask=mask0)

        for bb in range(2):
            b = wid * 2 + bb
            pltpu.sync_copy(tok_hbm.at[pl.ds(b * _IPAD, _IPAD)], idx_v)
            in_copy(b, 0, slab_a, s_ia).start()

            def i_body(i, c):
                dt_a = i * 2
                dt_b = dt_a + 1
                in_copy(b, dt_a, slab_a, s_ia).wait()
                in_copy(b, dt_b, slab_b, s_ib).start()

                @pl.when(i > 0)
                def _():
                    out_copy(b, dt_a - 2, oslab_a, s_oa).wait()

                compute(slab_a, oslab_a)
                out_copy(b, dt_a, oslab_a, s_oa).start()

                in_copy(b, dt_b, slab_b, s_ib).wait()
                nxt = jnp.minimum(dt_b + 1, _DT - 1)
                in_copy(b, nxt, slab_a, s_ia).start()

                @pl.when(i > 0)
                def _():
                    out_copy(b, dt_b - 2, oslab_b, s_ob).wait()

                compute(slab_b, oslab_b)
                out_copy(b, dt_b, oslab_b, s_ob).start()
                return c

            lax.fori_loop(0, _DT // 2, i_body, 0, unroll=False)
            in_copy(b, _DT - 1, slab_a, s_ia).wait()
            out_copy(b, _DT - 2, oslab_a, s_oa).wait()
            out_copy(b, _DT - 1, oslab_b, s_ob).wait()

    return gather_kernel


def kernel(x):
    out_t = _build()(x.transpose(0, 2, 1), _TOK)
    return out_t.transpose(0, 2, 1)
